# Initial kernel scaffold; baseline (speedup 1.0000x reference)
import jax
import jax.numpy as jnp
from jax.experimental import pallas as pl


def kernel(mask, gt, gdc_mask, gloss_mask, w_all, b_all):
    raise NotImplementedError("write your pallas kernel here")


# speedup vs baseline: 1.2950x; 1.2950x over previous
import functools

import numpy as np
import jax
import jax.numpy as jnp
from jax.experimental import pallas as pl
from jax.experimental.pallas import tpu as pltpu

_NL = 5
_TAPS = 9


def _dft_consts(H, W):
    def cs(n):
        k = np.arange(n)
        ang = -2.0 * np.pi * np.outer(k, k) / n
        return np.cos(ang) / np.sqrt(n), np.sin(ang) / np.sqrt(n)

    hr, hi = cs(H)
    wr, wi = cs(W)
    fwd = np.block([[hr, -hi], [hi, hr]])
    inv = np.block([[hr, hi], [-hi, hr]])
    col = np.concatenate([wr, wi], axis=1)
    return (jnp.asarray(fwd, jnp.float32), jnp.asarray(inv, jnp.float32),
            jnp.asarray(col, jnp.float32))


def _wrap_masks(H, W):
    j = np.arange(H * W) % W
    return jnp.asarray(np.stack([(j != 0), (j != W - 1)]).astype(np.float32))


def _cnn_dc_kernel(mhf_ref, mhi_ref, nw_ref, em_ref, w_ref, b_ref,
                   mask_ref, gdc_ref, gloss_ref, gt_ref,
                   out_ref, dcm_ref, lossm_ref,
                   xg_ref, p_ref, x_ref, k0_ref, dc_ref, *, H, W, G, F):
    L = H * W
    c = pl.program_id(1)
    mhf = mhf_ref[...]
    mhi = mhi_ref[...]
    nw = nw_ref[...]

    def fft2(z):
        a = jnp.dot(mhf, z, preferred_element_type=jnp.float32)
        p = jnp.dot(a, nw, preferred_element_type=jnp.float32)
        return jnp.concatenate(
            [p[0:H, 0:W] - p[H:, W:], p[0:H, W:] + p[H:, 0:W]], axis=0)

    def ifft2(z):
        a = jnp.dot(mhi, z, preferred_element_type=jnp.float32)
        p = jnp.dot(a, nw, preferred_element_type=jnp.float32)
        return jnp.concatenate(
            [p[0:H, 0:W] + p[H:, W:], p[H:, 0:W] - p[0:H, W:]], axis=0)

    @pl.when(c == 0)
    def _init():
        m = mask_ref[0, :, :]
        dc = gdc_ref[0, :, :] * m
        dc_ref[...] = dc
        dcm_ref[0, :, :] = dc
        lossm_ref[0, :, :] = gloss_ref[0, :, :] * m
        k0 = dc * fft2(gt_ref[0, :, :])
        k0_ref[...] = k0
        x_ref[...] = ifft2(k0)
        xg_ref[...] = jnp.zeros_like(xg_ref)

    xg_ref[0:2, G:G + L] = x_ref[...].reshape(2, L)

    ml = em_ref[0:1, :]
    mr = em_ref[1:2, :]
    h = None
    for l in range(_NL):
        for t in range(_TAPS):
            dkh = t // 3 - 1
            dkw = t % 3 - 1
            blk = xg_ref[:, G + dkh * W + dkw:G + dkh * W + dkw + L]
            if dkw == -1:
                blk = blk * ml
            elif dkw == 1:
                blk = blk * mr
            p_ref[t * F:(t + 1) * F, :] = blk
        acc = jnp.dot(w_ref[l], p_ref[...],
                      preferred_element_type=jnp.float32) + b_ref[l]
        if l < _NL - 1:
            xg_ref[:, G:G + L] = jnp.maximum(acc, 0.0)
        else:
            h = acc[0:2, :]

    x = x_ref[...] + h.reshape(2 * H, W)
    dc = dc_ref[...]
    y = ifft2((1.0 - dc) * fft2(x) + dc * k0_ref[...])
    x_ref[...] = y

    @pl.when(c == pl.num_programs(1) - 1)
    def _fin():
        out_ref[0, :, :] = y


def kernel(mask, gt, gdc_mask, gloss_mask, w_all, b_all):
    B, _, H, W = mask.shape
    L = H * W
    F = w_all.shape[1]
    G = ((W + 1 + 127) // 128) * 128
    nc = w_all.shape[0] // (_NL * _TAPS)

    mhf, mhi, nw = _dft_consts(H, W)
    em = _wrap_masks(H, W)

    w_pack = (w_all.reshape(nc, _NL, _TAPS, F, F)
              .transpose(0, 1, 3, 2, 4)
              .reshape(nc * _NL, F, _TAPS * F))

    mask_s = mask.reshape(B, 2 * H, W)
    gdc_s = gdc_mask.reshape(B, 2 * H, W)
    gloss_s = gloss_mask.reshape(B, 2 * H, W)
    gt_s = gt.reshape(B, 2 * H, W)

    img = pl.BlockSpec((1, 2 * H, W), lambda b, c: (b, 0, 0))

    def cst(shp):
        return pl.BlockSpec(shp, lambda b, c: (0,) * len(shp))

    osd = jax.ShapeDtypeStruct((B, 2 * H, W), jnp.float32)
    body = functools.partial(_cnn_dc_kernel, H=H, W=W, G=G, F=F)

    out_s, dcm_s, lossm_s = pl.pallas_call(
        body,
        out_shape=(osd, osd, osd),
        grid=(B, nc),
        in_specs=[
            cst((2 * H, 2 * H)),
            cst((2 * H, 2 * H)),
            cst((W, 2 * W)),
            cst((2, L)),
            pl.BlockSpec((_NL, F, _TAPS * F), lambda b, c: (c, 0, 0)),
            pl.BlockSpec((_NL, F, 1), lambda b, c: (c, 0, 0)),
            img, img, img, img,
        ],
        out_specs=(img, img, img),
        scratch_shapes=[
            pltpu.VMEM((F, L + 2 * G), jnp.float32),
            pltpu.VMEM((_TAPS * F, L), jnp.float32),
            pltpu.VMEM((2 * H, W), jnp.float32),
            pltpu.VMEM((2 * H, W), jnp.float32),
            pltpu.VMEM((2 * H, W), jnp.float32),
        ],
        compiler_params=pltpu.CompilerParams(
            dimension_semantics=("parallel", "arbitrary"),
            vmem_limit_bytes=64 * 1024 * 1024),
    )(mhf, mhi, nw, em, w_pack, b_all, mask_s, gdc_s, gloss_s, gt_s)

    return (out_s.reshape(B, 2, H, W),
            lossm_s.reshape(B, 2, H, W),
            dcm_s.reshape(B, 2, H, W),
            mask)
